# Initial kernel scaffold; baseline (speedup 1.0000x reference)
import jax
import jax.numpy as jnp
from jax.experimental import pallas as pl


def kernel(x, edge_index, edge_label_index, W1, b1, W2, b2, W3, b3, Wp, bp):
    raise NotImplementedError("write your pallas kernel here")


# speedup vs baseline: 3.0760x; 3.0760x over previous
import functools

import jax
import jax.numpy as jnp
from jax import lax
from jax.experimental import pallas as pl
from jax.experimental.pallas import tpu as pltpu
from jax.experimental.pallas import tpu_sc as plsc

N = 10000
NP = 10240
D = 128
NC, NS = 2, 16
CHUNK = 128

E_CHUNKS = 2560
E_PAD = E_CHUNKS * CHUNK
CPT = E_CHUNKS // NS
IG = 32

L_CHUNKS = 1024
L_PAD = L_CHUNKS * CHUNK
LCPT = L_CHUNKS // (NC * NS)

ROWS_PT = NP // NS

_ROW_CHUNKS = [(i * 128, 128) for i in range(5)]

_MESH = dict(core_axis_name="c", subcore_axis_name="s")


@functools.partial(
    pl.kernel,
    out_type=jax.ShapeDtypeStruct((2 * NP, D), jnp.float32),
    mesh=plsc.VectorSubcoreMesh(**_MESH),
    scratch_types=[
        pltpu.VMEM((IG, CHUNK), jnp.int32),
        pltpu.VMEM((IG, CHUNK), jnp.int32),
        pltpu.VMEM((CHUNK, D), jnp.float32),
        pltpu.VMEM_SHARED((NP, D), jnp.float32),
        pltpu.SemaphoreType.DMA,
    ],
)
def _agg(h_hbm, srcc_hbm, dst_hbm, agg_hbm, si_v, di_v, gbuf, agg_sh, sem):
    c = lax.axis_index("c")
    s = lax.axis_index("s")
    base = c * NP
    row0 = s * ROWS_PT

    for off, sz in _ROW_CHUNKS:
        pltpu.sync_copy(h_hbm.at[pl.ds(base + row0 + off, sz)],
                        gbuf.at[pl.ds(0, sz)])
        pltpu.sync_copy(gbuf.at[pl.ds(0, sz)],
                        agg_sh.at[pl.ds(row0 + off, sz)])
    plsc.subcore_barrier()

    def outer(g, carry):
        pltpu.sync_copy(
            srcc_hbm.at[pl.ds(c * E_CHUNKS + s * CPT + g * IG, IG)], si_v)
        pltpu.sync_copy(dst_hbm.at[pl.ds(s * CPT + g * IG, IG)], di_v)

        def body(j, carry2):
            pltpu.async_copy(h_hbm.at[si_v.at[j]], gbuf, sem).wait()
            pltpu.sync_copy(gbuf, agg_sh.at[di_v.at[j]], add=True)
            return carry2

        lax.fori_loop(0, IG, body, 0)
        return carry

    lax.fori_loop(0, CPT // IG, outer, 0)
    plsc.subcore_barrier()

    for off, sz in _ROW_CHUNKS:
        pltpu.sync_copy(agg_sh.at[pl.ds(row0 + off, sz)],
                        gbuf.at[pl.ds(0, sz)])
        pltpu.sync_copy(gbuf.at[pl.ds(0, sz)],
                        agg_hbm.at[pl.ds(base + row0 + off, sz)])


@functools.partial(
    pl.kernel,
    out_type=(jax.ShapeDtypeStruct((L_PAD, D), jnp.float32),
              jax.ShapeDtypeStruct((L_PAD, D), jnp.float32)),
    mesh=plsc.VectorSubcoreMesh(**_MESH),
    scratch_types=[
        pltpu.VMEM((LCPT, CHUNK), jnp.int32),
        pltpu.VMEM((LCPT, CHUNK), jnp.int32),
        pltpu.VMEM((CHUNK, D), jnp.float32),
        pltpu.VMEM((CHUNK, D), jnp.float32),
        pltpu.SemaphoreType.DMA,
        pltpu.SemaphoreType.DMA,
    ],
)
def _label_kernel(t0_hbm, t1_hbm, e0_hbm, e1_hbm, hs_hbm, hd_hbm,
                  i0, i1, g0, g1, s0, s1):
    c = lax.axis_index("c")
    s = lax.axis_index("s")
    wid = s * NC + c
    pltpu.sync_copy(e0_hbm.at[pl.ds(wid * LCPT, LCPT)], i0)
    pltpu.sync_copy(e1_hbm.at[pl.ds(wid * LCPT, LCPT)], i1)

    def body(j, carry):
        cp0 = pltpu.async_copy(t0_hbm.at[i0.at[j]], g0, s0)
        cp1 = pltpu.async_copy(t1_hbm.at[i1.at[j]], g1, s1)
        cp0.wait()
        cp1.wait()
        row = wid * (LCPT * CHUNK) + j * CHUNK
        pltpu.sync_copy(g0, hs_hbm.at[pl.ds(row, CHUNK)])
        pltpu.sync_copy(g1, hd_hbm.at[pl.ds(row, CHUNK)])
        return carry

    lax.fori_loop(0, LCPT, body, 0)


R = 1024
NR = NP // R
RL = 1024
NL = L_PAD // RL


def _row_mask(r):
    ridx = lax.broadcasted_iota(jnp.int32, (R, 1), 0) + r * R
    return ridx < N


def _c1_body(x_ref, w_ref, deg_ref, o_ref):
    dinv = lax.rsqrt(deg_ref[:, 0:1])
    h = jnp.dot(x_ref[...] * dinv, w_ref[...],
                preferred_element_type=jnp.float32)
    o_ref[...] = jnp.where(_row_mask(pl.program_id(1)), h, 0.0)


_c1 = pl.pallas_call(
    _c1_body,
    grid=(2, NR),
    in_specs=[pl.BlockSpec((R, D), lambda c, r: (r, 0)),
              pl.BlockSpec((D, D), lambda c, r: (0, c)),
              pl.BlockSpec((R, D), lambda c, r: (r, 0))],
    out_specs=pl.BlockSpec((R, D), lambda c, r: (c * NR + r, 0)),
    out_shape=jax.ShapeDtypeStruct((2 * NP, D), jnp.float32),
)


def _mid_body(lo_ref, hi_ref, w_ref, b_ref, deg_ref, o_ref):
    dinv = lax.rsqrt(deg_ref[:, 0:1])
    lo = lo_ref[...] * dinv + b_ref[0:1, :]
    hi = hi_ref[...] * dinv + b_ref[1:2, :]
    w = w_ref[0]
    h = (jnp.dot(lo, w[:D, :], preferred_element_type=jnp.float32)
         + jnp.dot(hi, w[D:, :], preferred_element_type=jnp.float32))
    o_ref[...] = jnp.where(_row_mask(pl.program_id(1)), h * dinv, 0.0)


def _make_mid(w_im):
    return pl.pallas_call(
        _mid_body,
        grid=(2, NR),
        in_specs=[pl.BlockSpec((R, D), lambda c, r: (r, 0)),
                  pl.BlockSpec((R, D), lambda c, r: (NR + r, 0)),
                  pl.BlockSpec((1, 2 * D, D), w_im),
                  pl.BlockSpec((2, D), lambda c, r: (0, 0)),
                  pl.BlockSpec((R, D), lambda c, r: (r, 0))],
        out_specs=pl.BlockSpec((R, D), lambda c, r: (c * NR + r, 0)),
        out_shape=jax.ShapeDtypeStruct((2 * NP, D), jnp.float32),
    )


_c2 = _make_mid(lambda c, r: (c, 0, 0))
_c3 = _make_mid(lambda c, r: (0, 0, 0))


def _c4_body(a_ref, deg_ref, b_ref, wp_ref, o3_ref, o3w_ref):
    dinv = lax.rsqrt(deg_ref[:, 0:1])
    t = a_ref[...] * dinv + b_ref[...]
    o3_ref[...] = t
    o3w_ref[...] = t * wp_ref[...]


_c4 = pl.pallas_call(
    _c4_body,
    grid=(NR,),
    in_specs=[pl.BlockSpec((R, D), lambda r: (r, 0)),
              pl.BlockSpec((R, D), lambda r: (r, 0)),
              pl.BlockSpec((1, D), lambda r: (0, 0)),
              pl.BlockSpec((1, D), lambda r: (0, 0))],
    out_specs=(pl.BlockSpec((R, D), lambda r: (r, 0)),
               pl.BlockSpec((R, D), lambda r: (r, 0))),
    out_shape=(jax.ShapeDtypeStruct((NP, D), jnp.float32),
               jax.ShapeDtypeStruct((NP, D), jnp.float32)),
)


def _c5_body(a_ref, b_ref, bp_ref, o_ref):
    o_ref[...] = (jnp.sum(a_ref[...] * b_ref[...], axis=1)[None, None, :]
                  + bp_ref[0, 0])


_c5 = pl.pallas_call(
    _c5_body,
    grid=(NL,),
    in_specs=[pl.BlockSpec((RL, D), lambda r: (r, 0)),
              pl.BlockSpec((RL, D), lambda r: (r, 0)),
              pl.BlockSpec((1, 1), lambda r: (0, 0))],
    out_specs=pl.BlockSpec((1, 1, RL), lambda r: (r, 0, 0)),
    out_shape=jax.ShapeDtypeStruct((NL, 1, RL), jnp.float32),
)


def kernel(x, edge_index, edge_label_index, W1, b1, W2, b2, W3, b3, Wp, bp):
    src = edge_index[0].astype(jnp.int32)
    dst = edge_index[1].astype(jnp.int32)
    n_e = src.shape[0]
    src_p = jnp.concatenate([src, jnp.full((E_PAD - n_e,), N, jnp.int32)])
    dst_p = jnp.concatenate([dst, jnp.zeros((E_PAD - n_e,), jnp.int32)])
    src2d = src_p.reshape(E_CHUNKS, CHUNK)
    srcc = jnp.concatenate([src2d, src2d + NP], axis=0)
    dst2d = dst_p.reshape(E_CHUNKS, CHUNK)

    e0 = edge_label_index[0].astype(jnp.int32)
    e1 = edge_label_index[1].astype(jnp.int32)
    n_l = e0.shape[0]
    e0_2d = jnp.concatenate(
        [e0, jnp.zeros((L_PAD - n_l,), jnp.int32)]).reshape(L_CHUNKS, CHUNK)
    e1_2d = jnp.concatenate(
        [e1, jnp.zeros((L_PAD - n_l,), jnp.int32)]).reshape(L_CHUNKS, CHUNK)

    ones_half = jnp.concatenate(
        [jnp.ones((N, D), jnp.float32),
         jnp.zeros((NP - N, D), jnp.float32)], axis=0)
    ones_tab = jnp.concatenate([ones_half, ones_half], axis=0)
    deg = _agg(ones_tab, srcc, dst2d)

    x_p = jnp.concatenate(
        [x, jnp.zeros((NP - x.shape[0], D), jnp.float32)], axis=0)
    w2s = W2.reshape(2 * D, 2, D).transpose(1, 0, 2)
    w3s = W3.reshape(1, 2 * D, D)

    h1 = _c1(x_p, W1, deg)
    agg1 = _agg(h1, srcc, dst2d)
    h2 = _c2(agg1, agg1, w2s, b1.reshape(2, D), deg)
    agg2 = _agg(h2, srcc, dst2d)
    h3 = _c3(agg2, agg2, w3s, b2.reshape(2, D), deg)
    agg3 = _agg(h3, srcc, dst2d)
    out3, out3w = _c4(agg3, deg, b3.reshape(1, D), Wp.reshape(1, D))
    hs, hd = _label_kernel(out3w, out3, e0_2d, e1_2d)
    pred3d = _c5(hs, hd, bp.reshape(1, 1))
    return pred3d.reshape(L_PAD)[:n_l]


# speedup vs baseline: 3.4307x; 1.1153x over previous
import functools

import jax
import jax.numpy as jnp
from jax import lax
from jax.experimental import pallas as pl
from jax.experimental.pallas import tpu as pltpu
from jax.experimental.pallas import tpu_sc as plsc

N = 10000
NP = 10240
D = 128
NC, NS = 2, 16
CHUNK = 128

E_CHUNKS = 2560
E_PAD = E_CHUNKS * CHUNK
CPT = E_CHUNKS // NS
IG = 16

L_CHUNKS = 1024
L_PAD = L_CHUNKS * CHUNK
LCPT = L_CHUNKS // (NC * NS)

ROWS_PT = NP // NS

_ROW_CHUNKS = [(i * 128, 128) for i in range(5)]

_MESH = dict(core_axis_name="c", subcore_axis_name="s")


@functools.partial(
    pl.kernel,
    out_type=jax.ShapeDtypeStruct((2 * NP, D), jnp.float32),
    mesh=plsc.VectorSubcoreMesh(**_MESH),
    scratch_types=[
        pltpu.VMEM((IG, CHUNK), jnp.int32),
        pltpu.VMEM((IG, CHUNK), jnp.int32),
        pltpu.VMEM((CHUNK, D), jnp.float32),
        pltpu.VMEM((CHUNK, D), jnp.float32),
        pltpu.VMEM_SHARED((NP, D), jnp.float32),
        pltpu.SemaphoreType.DMA,
        pltpu.SemaphoreType.DMA,
        pltpu.SemaphoreType.DMA,
        pltpu.SemaphoreType.DMA,
    ],
)
def _agg(h_hbm, srcc_hbm, dst_hbm, agg_hbm, si_v, di_v, gb0, gb1, agg_sh,
         sg0, sg1, ss0, ss1):
    c = lax.axis_index("c")
    s = lax.axis_index("s")
    base = c * NP
    row0 = s * ROWS_PT

    for off, sz in _ROW_CHUNKS:
        pltpu.sync_copy(h_hbm.at[pl.ds(base + row0 + off, sz)],
                        gb0.at[pl.ds(0, sz)])
        pltpu.sync_copy(gb0.at[pl.ds(0, sz)],
                        agg_sh.at[pl.ds(row0 + off, sz)])
    plsc.subcore_barrier()

    bufs = (gb0, gb1)
    sgs = (sg0, sg1)
    sss = (ss0, ss1)

    def group(g, carry):
        pltpu.sync_copy(
            srcc_hbm.at[pl.ds(c * E_CHUNKS + s * CPT + g * IG, IG)], si_v)
        pltpu.sync_copy(dst_hbm.at[pl.ds(s * CPT + g * IG, IG)], di_v)

        cpg = [None] * IG
        cps = [None] * IG
        cpg[0] = pltpu.async_copy(h_hbm.at[si_v.at[0]], bufs[0], sgs[0])
        for j in range(IG):
            p = j % 2
            if j >= 1:
                cps[j - 1].wait()
            if j + 1 < IG:
                cpg[j + 1] = pltpu.async_copy(
                    h_hbm.at[si_v.at[j + 1]], bufs[1 - p], sgs[1 - p])
            cpg[j].wait()
            cps[j] = pltpu.async_copy(
                bufs[p], agg_sh.at[di_v.at[j]], sss[p], add=True)
        cps[IG - 1].wait()
        return carry

    lax.fori_loop(0, CPT // IG, group, 0)
    plsc.subcore_barrier()

    for off, sz in _ROW_CHUNKS:
        pltpu.sync_copy(agg_sh.at[pl.ds(row0 + off, sz)],
                        gb0.at[pl.ds(0, sz)])
        pltpu.sync_copy(gb0.at[pl.ds(0, sz)],
                        agg_hbm.at[pl.ds(base + row0 + off, sz)])


@functools.partial(
    pl.kernel,
    out_type=(jax.ShapeDtypeStruct((L_PAD, D), jnp.float32),
              jax.ShapeDtypeStruct((L_PAD, D), jnp.float32)),
    mesh=plsc.VectorSubcoreMesh(**_MESH),
    scratch_types=[
        pltpu.VMEM((LCPT, CHUNK), jnp.int32),
        pltpu.VMEM((LCPT, CHUNK), jnp.int32),
        pltpu.VMEM((CHUNK, D), jnp.float32),
        pltpu.VMEM((CHUNK, D), jnp.float32),
        pltpu.SemaphoreType.DMA,
        pltpu.SemaphoreType.DMA,
    ],
)
def _label_kernel(t0_hbm, t1_hbm, e0_hbm, e1_hbm, hs_hbm, hd_hbm,
                  i0, i1, g0, g1, s0, s1):
    c = lax.axis_index("c")
    s = lax.axis_index("s")
    wid = s * NC + c
    pltpu.sync_copy(e0_hbm.at[pl.ds(wid * LCPT, LCPT)], i0)
    pltpu.sync_copy(e1_hbm.at[pl.ds(wid * LCPT, LCPT)], i1)

    def body(j, carry):
        cp0 = pltpu.async_copy(t0_hbm.at[i0.at[j]], g0, s0)
        cp1 = pltpu.async_copy(t1_hbm.at[i1.at[j]], g1, s1)
        cp0.wait()
        cp1.wait()
        row = wid * (LCPT * CHUNK) + j * CHUNK
        pltpu.sync_copy(g0, hs_hbm.at[pl.ds(row, CHUNK)])
        pltpu.sync_copy(g1, hd_hbm.at[pl.ds(row, CHUNK)])
        return carry

    lax.fori_loop(0, LCPT, body, 0)


R = 1024
NR = NP // R
RL = 1024
NL = L_PAD // RL


def _row_mask(r):
    ridx = lax.broadcasted_iota(jnp.int32, (R, 1), 0) + r * R
    return ridx < N


def _c1_body(x_ref, w_ref, deg_ref, o_ref):
    dinv = lax.rsqrt(deg_ref[:, 0:1])
    h = jnp.dot(x_ref[...] * dinv, w_ref[...],
                preferred_element_type=jnp.float32)
    o_ref[...] = jnp.where(_row_mask(pl.program_id(1)), h, 0.0)


_c1 = pl.pallas_call(
    _c1_body,
    grid=(2, NR),
    in_specs=[pl.BlockSpec((R, D), lambda c, r: (r, 0)),
              pl.BlockSpec((D, D), lambda c, r: (0, c)),
              pl.BlockSpec((R, D), lambda c, r: (r, 0))],
    out_specs=pl.BlockSpec((R, D), lambda c, r: (c * NR + r, 0)),
    out_shape=jax.ShapeDtypeStruct((2 * NP, D), jnp.float32),
)


def _mid_body(lo_ref, hi_ref, w_ref, b_ref, deg_ref, o_ref):
    dinv = lax.rsqrt(deg_ref[:, 0:1])
    lo = lo_ref[...] * dinv + b_ref[0:1, :]
    hi = hi_ref[...] * dinv + b_ref[1:2, :]
    w = w_ref[0]
    h = (jnp.dot(lo, w[:D, :], preferred_element_type=jnp.float32)
         + jnp.dot(hi, w[D:, :], preferred_element_type=jnp.float32))
    o_ref[...] = jnp.where(_row_mask(pl.program_id(1)), h * dinv, 0.0)


def _make_mid(w_im):
    return pl.pallas_call(
        _mid_body,
        grid=(2, NR),
        in_specs=[pl.BlockSpec((R, D), lambda c, r: (r, 0)),
                  pl.BlockSpec((R, D), lambda c, r: (NR + r, 0)),
                  pl.BlockSpec((1, 2 * D, D), w_im),
                  pl.BlockSpec((2, D), lambda c, r: (0, 0)),
                  pl.BlockSpec((R, D), lambda c, r: (r, 0))],
        out_specs=pl.BlockSpec((R, D), lambda c, r: (c * NR + r, 0)),
        out_shape=jax.ShapeDtypeStruct((2 * NP, D), jnp.float32),
    )


_c2 = _make_mid(lambda c, r: (c, 0, 0))
_c3 = _make_mid(lambda c, r: (0, 0, 0))


def _c4_body(a_ref, deg_ref, b_ref, wp_ref, o3_ref, o3w_ref):
    dinv = lax.rsqrt(deg_ref[:, 0:1])
    t = a_ref[...] * dinv + b_ref[...]
    o3_ref[...] = t
    o3w_ref[...] = t * wp_ref[...]


_c4 = pl.pallas_call(
    _c4_body,
    grid=(NR,),
    in_specs=[pl.BlockSpec((R, D), lambda r: (r, 0)),
              pl.BlockSpec((R, D), lambda r: (r, 0)),
              pl.BlockSpec((1, D), lambda r: (0, 0)),
              pl.BlockSpec((1, D), lambda r: (0, 0))],
    out_specs=(pl.BlockSpec((R, D), lambda r: (r, 0)),
               pl.BlockSpec((R, D), lambda r: (r, 0))),
    out_shape=(jax.ShapeDtypeStruct((NP, D), jnp.float32),
               jax.ShapeDtypeStruct((NP, D), jnp.float32)),
)


def _c5_body(a_ref, b_ref, bp_ref, o_ref):
    o_ref[...] = (jnp.sum(a_ref[...] * b_ref[...], axis=1)[None, None, :]
                  + bp_ref[0, 0])


_c5 = pl.pallas_call(
    _c5_body,
    grid=(NL,),
    in_specs=[pl.BlockSpec((RL, D), lambda r: (r, 0)),
              pl.BlockSpec((RL, D), lambda r: (r, 0)),
              pl.BlockSpec((1, 1), lambda r: (0, 0))],
    out_specs=pl.BlockSpec((1, 1, RL), lambda r: (r, 0, 0)),
    out_shape=jax.ShapeDtypeStruct((NL, 1, RL), jnp.float32),
)


def kernel(x, edge_index, edge_label_index, W1, b1, W2, b2, W3, b3, Wp, bp):
    src = edge_index[0].astype(jnp.int32)
    dst = edge_index[1].astype(jnp.int32)
    n_e = src.shape[0]
    src_p = jnp.concatenate([src, jnp.full((E_PAD - n_e,), N, jnp.int32)])
    dst_p = jnp.concatenate([dst, jnp.zeros((E_PAD - n_e,), jnp.int32)])
    src2d = src_p.reshape(E_CHUNKS, CHUNK)
    srcc = jnp.concatenate([src2d, src2d + NP], axis=0)
    dst2d = dst_p.reshape(E_CHUNKS, CHUNK)

    e0 = edge_label_index[0].astype(jnp.int32)
    e1 = edge_label_index[1].astype(jnp.int32)
    n_l = e0.shape[0]
    e0_2d = jnp.concatenate(
        [e0, jnp.zeros((L_PAD - n_l,), jnp.int32)]).reshape(L_CHUNKS, CHUNK)
    e1_2d = jnp.concatenate(
        [e1, jnp.zeros((L_PAD - n_l,), jnp.int32)]).reshape(L_CHUNKS, CHUNK)

    ones_half = jnp.concatenate(
        [jnp.ones((N, D), jnp.float32),
         jnp.zeros((NP - N, D), jnp.float32)], axis=0)
    ones_tab = jnp.concatenate([ones_half, ones_half], axis=0)
    deg = _agg(ones_tab, srcc, dst2d)

    x_p = jnp.concatenate(
        [x, jnp.zeros((NP - x.shape[0], D), jnp.float32)], axis=0)
    w2s = W2.reshape(2 * D, 2, D).transpose(1, 0, 2)
    w3s = W3.reshape(1, 2 * D, D)

    h1 = _c1(x_p, W1, deg)
    agg1 = _agg(h1, srcc, dst2d)
    h2 = _c2(agg1, agg1, w2s, b1.reshape(2, D), deg)
    agg2 = _agg(h2, srcc, dst2d)
    h3 = _c3(agg2, agg2, w3s, b2.reshape(2, D), deg)
    agg3 = _agg(h3, srcc, dst2d)
    out3, out3w = _c4(agg3, deg, b3.reshape(1, D), Wp.reshape(1, D))
    hs, hd = _label_kernel(out3w, out3, e0_2d, e1_2d)
    pred3d = _c5(hs, hd, bp.reshape(1, 1))
    return pred3d.reshape(L_PAD)[:n_l]


# speedup vs baseline: 3.4449x; 1.0041x over previous
import functools

import jax
import jax.numpy as jnp
from jax import lax
from jax.experimental import pallas as pl
from jax.experimental.pallas import tpu as pltpu
from jax.experimental.pallas import tpu_sc as plsc

N = 10000
NP = 10240
D = 128
NC, NS = 2, 16
CHUNK = 128

E_CHUNKS = 2560
E_PAD = E_CHUNKS * CHUNK
CPT = E_CHUNKS // NS
IG = 16

L_CHUNKS = 1024
L_PAD = L_CHUNKS * CHUNK
LCPT = L_CHUNKS // (NC * NS)

ROWS_PT = NP // NS

_ROW_CHUNKS = [(i * 128, 128) for i in range(5)]

_MESH = dict(core_axis_name="c", subcore_axis_name="s")


@functools.partial(
    pl.kernel,
    out_type=jax.ShapeDtypeStruct((2 * NP, D), jnp.float32),
    mesh=plsc.VectorSubcoreMesh(**_MESH),
    scratch_types=[
        pltpu.VMEM((IG, CHUNK), jnp.int32),
        pltpu.VMEM((IG, CHUNK), jnp.int32),
        pltpu.VMEM((CHUNK, D), jnp.float32),
        pltpu.VMEM((CHUNK, D), jnp.float32),
        pltpu.VMEM_SHARED((NP, D), jnp.float32),
        pltpu.SemaphoreType.DMA,
        pltpu.SemaphoreType.DMA,
        pltpu.SemaphoreType.DMA,
        pltpu.SemaphoreType.DMA,
    ],
)
def _agg(h_hbm, srcc_hbm, dst_hbm, agg_hbm, si_v, di_v, gb0, gb1, agg_sh,
         sg0, sg1, ss0, ss1):
    c = lax.axis_index("c")
    s = lax.axis_index("s")
    base = c * NP
    row0 = s * ROWS_PT

    for off, sz in _ROW_CHUNKS:
        pltpu.sync_copy(h_hbm.at[pl.ds(base + row0 + off, sz)],
                        gb0.at[pl.ds(0, sz)])
        pltpu.sync_copy(gb0.at[pl.ds(0, sz)],
                        agg_sh.at[pl.ds(row0 + off, sz)])
    plsc.subcore_barrier()

    bufs = (gb0, gb1)
    sgs = (sg0, sg1)
    sss = (ss0, ss1)

    def group(g, carry):
        pltpu.sync_copy(
            srcc_hbm.at[pl.ds(c * E_CHUNKS + s * CPT + g * IG, IG)], si_v)
        pltpu.sync_copy(dst_hbm.at[pl.ds(s * CPT + g * IG, IG)], di_v)

        cpg = [None] * IG
        cps = [None] * IG
        cpg[0] = pltpu.async_copy(h_hbm.at[si_v.at[0]], bufs[0], sgs[0])
        for j in range(IG):
            p = j % 2
            if j >= 1:
                cps[j - 1].wait()
            if j + 1 < IG:
                cpg[j + 1] = pltpu.async_copy(
                    h_hbm.at[si_v.at[j + 1]], bufs[1 - p], sgs[1 - p])
            cpg[j].wait()
            cps[j] = pltpu.async_copy(
                bufs[p], agg_sh.at[di_v.at[j]], sss[p], add=True)
        cps[IG - 1].wait()
        return carry

    lax.fori_loop(0, CPT // IG, group, 0)
    plsc.subcore_barrier()

    for off, sz in _ROW_CHUNKS:
        pltpu.sync_copy(agg_sh.at[pl.ds(row0 + off, sz)],
                        gb0.at[pl.ds(0, sz)])
        pltpu.sync_copy(gb0.at[pl.ds(0, sz)],
                        agg_hbm.at[pl.ds(base + row0 + off, sz)])


@functools.partial(
    pl.kernel,
    out_type=(jax.ShapeDtypeStruct((L_PAD, D), jnp.float32),
              jax.ShapeDtypeStruct((L_PAD, D), jnp.float32)),
    mesh=plsc.VectorSubcoreMesh(**_MESH),
    scratch_types=[
        pltpu.VMEM((LCPT, CHUNK), jnp.int32),
        pltpu.VMEM((LCPT, CHUNK), jnp.int32),
        pltpu.VMEM((CHUNK, D), jnp.float32),
        pltpu.VMEM((CHUNK, D), jnp.float32),
        pltpu.VMEM((CHUNK, D), jnp.float32),
        pltpu.VMEM((CHUNK, D), jnp.float32),
        pltpu.VMEM((CHUNK, D), jnp.float32),
        pltpu.VMEM((CHUNK, D), jnp.float32),
        pltpu.SemaphoreType.DMA,
        pltpu.SemaphoreType.DMA,
        pltpu.SemaphoreType.DMA,
        pltpu.SemaphoreType.DMA,
        pltpu.SemaphoreType.DMA,
        pltpu.SemaphoreType.DMA,
    ],
)
def _label_kernel(t0_hbm, t1_hbm, e0_hbm, e1_hbm, hs_hbm, hd_hbm,
                  i0, i1, a0, a1, a2, b0, b1, b2, sa0, sa1, sa2,
                  sb0, sb1, sb2):
    c = lax.axis_index("c")
    s = lax.axis_index("s")
    wid = s * NC + c
    pltpu.sync_copy(e0_hbm.at[pl.ds(wid * LCPT, LCPT)], i0)
    pltpu.sync_copy(e1_hbm.at[pl.ds(wid * LCPT, LCPT)], i1)

    slots_a = (a0, a1, a2)
    slots_b = (b0, b1, b2)
    sems_a = (sa0, sa1, sa2)
    sems_b = (sb0, sb1, sb2)

    def g_issue(j):
        p = j % 3
        ca = pltpu.async_copy(t0_hbm.at[i0.at[j]], slots_a[p], sems_a[p])
        cb = pltpu.async_copy(t1_hbm.at[i1.at[j]], slots_b[p], sems_b[p])
        return ca, cb

    cg = [None] * LCPT
    cw = [None] * LCPT
    cg[0] = g_issue(0)
    cg[1] = g_issue(1)
    for j in range(LCPT):
        p = j % 3
        if j >= 1:
            cw[j - 1][0].wait()
            cw[j - 1][1].wait()
        if j + 2 < LCPT:
            cg[j + 2] = g_issue(j + 2)
        cg[j][0].wait()
        cg[j][1].wait()
        row = wid * (LCPT * CHUNK) + j * CHUNK
        cw[j] = (pltpu.async_copy(slots_a[p], hs_hbm.at[pl.ds(row, CHUNK)],
                                  sems_a[p]),
                 pltpu.async_copy(slots_b[p], hd_hbm.at[pl.ds(row, CHUNK)],
                                  sems_b[p]))
    cw[LCPT - 1][0].wait()
    cw[LCPT - 1][1].wait()


R = 1024
NR = NP // R
RL = 1024
NL = L_PAD // RL


def _row_mask(r):
    ridx = lax.broadcasted_iota(jnp.int32, (R, 1), 0) + r * R
    return ridx < N


def _c1_body(x_ref, w_ref, deg_ref, o_ref):
    dinv = lax.rsqrt(deg_ref[:, 0:1])
    h = jnp.dot(x_ref[...] * dinv, w_ref[...],
                preferred_element_type=jnp.float32)
    o_ref[...] = jnp.where(_row_mask(pl.program_id(1)), h, 0.0)


_c1 = pl.pallas_call(
    _c1_body,
    grid=(2, NR),
    in_specs=[pl.BlockSpec((R, D), lambda c, r: (r, 0)),
              pl.BlockSpec((D, D), lambda c, r: (0, c)),
              pl.BlockSpec((R, D), lambda c, r: (r, 0))],
    out_specs=pl.BlockSpec((R, D), lambda c, r: (c * NR + r, 0)),
    out_shape=jax.ShapeDtypeStruct((2 * NP, D), jnp.float32),
)


def _mid_body(lo_ref, hi_ref, w_ref, b_ref, deg_ref, o_ref):
    dinv = lax.rsqrt(deg_ref[:, 0:1])
    lo = lo_ref[...] * dinv + b_ref[0:1, :]
    hi = hi_ref[...] * dinv + b_ref[1:2, :]
    w = w_ref[0]
    h = (jnp.dot(lo, w[:D, :], preferred_element_type=jnp.float32)
         + jnp.dot(hi, w[D:, :], preferred_element_type=jnp.float32))
    o_ref[...] = jnp.where(_row_mask(pl.program_id(1)), h * dinv, 0.0)


def _make_mid(w_im):
    return pl.pallas_call(
        _mid_body,
        grid=(2, NR),
        in_specs=[pl.BlockSpec((R, D), lambda c, r: (r, 0)),
                  pl.BlockSpec((R, D), lambda c, r: (NR + r, 0)),
                  pl.BlockSpec((1, 2 * D, D), w_im),
                  pl.BlockSpec((2, D), lambda c, r: (0, 0)),
                  pl.BlockSpec((R, D), lambda c, r: (r, 0))],
        out_specs=pl.BlockSpec((R, D), lambda c, r: (c * NR + r, 0)),
        out_shape=jax.ShapeDtypeStruct((2 * NP, D), jnp.float32),
    )


_c2 = _make_mid(lambda c, r: (c, 0, 0))
_c3 = _make_mid(lambda c, r: (0, 0, 0))


def _c4_body(a_ref, deg_ref, b_ref, wp_ref, o3_ref, o3w_ref):
    dinv = lax.rsqrt(deg_ref[:, 0:1])
    t = a_ref[...] * dinv + b_ref[...]
    o3_ref[...] = t
    o3w_ref[...] = t * wp_ref[...]


_c4 = pl.pallas_call(
    _c4_body,
    grid=(NR,),
    in_specs=[pl.BlockSpec((R, D), lambda r: (r, 0)),
              pl.BlockSpec((R, D), lambda r: (r, 0)),
              pl.BlockSpec((1, D), lambda r: (0, 0)),
              pl.BlockSpec((1, D), lambda r: (0, 0))],
    out_specs=(pl.BlockSpec((R, D), lambda r: (r, 0)),
               pl.BlockSpec((R, D), lambda r: (r, 0))),
    out_shape=(jax.ShapeDtypeStruct((NP, D), jnp.float32),
               jax.ShapeDtypeStruct((NP, D), jnp.float32)),
)


def _c5_body(a_ref, b_ref, bp_ref, o_ref):
    o_ref[...] = (jnp.sum(a_ref[...] * b_ref[...], axis=1)[None, None, :]
                  + bp_ref[0, 0])


_c5 = pl.pallas_call(
    _c5_body,
    grid=(NL,),
    in_specs=[pl.BlockSpec((RL, D), lambda r: (r, 0)),
              pl.BlockSpec((RL, D), lambda r: (r, 0)),
              pl.BlockSpec((1, 1), lambda r: (0, 0))],
    out_specs=pl.BlockSpec((1, 1, RL), lambda r: (r, 0, 0)),
    out_shape=jax.ShapeDtypeStruct((NL, 1, RL), jnp.float32),
)


def kernel(x, edge_index, edge_label_index, W1, b1, W2, b2, W3, b3, Wp, bp):
    src = edge_index[0].astype(jnp.int32)
    dst = edge_index[1].astype(jnp.int32)
    n_e = src.shape[0]
    src_p = jnp.concatenate([src, jnp.full((E_PAD - n_e,), N, jnp.int32)])
    dst_p = jnp.concatenate([dst, jnp.zeros((E_PAD - n_e,), jnp.int32)])
    src2d = src_p.reshape(E_CHUNKS, CHUNK)
    srcc = jnp.concatenate([src2d, src2d + NP], axis=0)
    dst2d = dst_p.reshape(E_CHUNKS, CHUNK)

    e0 = edge_label_index[0].astype(jnp.int32)
    e1 = edge_label_index[1].astype(jnp.int32)
    n_l = e0.shape[0]
    e0_2d = jnp.concatenate(
        [e0, jnp.zeros((L_PAD - n_l,), jnp.int32)]).reshape(L_CHUNKS, CHUNK)
    e1_2d = jnp.concatenate(
        [e1, jnp.zeros((L_PAD - n_l,), jnp.int32)]).reshape(L_CHUNKS, CHUNK)

    ones_half = jnp.concatenate(
        [jnp.ones((N, D), jnp.float32),
         jnp.zeros((NP - N, D), jnp.float32)], axis=0)
    ones_tab = jnp.concatenate([ones_half, ones_half], axis=0)
    deg = _agg(ones_tab, srcc, dst2d)

    x_p = jnp.concatenate(
        [x, jnp.zeros((NP - x.shape[0], D), jnp.float32)], axis=0)
    w2s = W2.reshape(2 * D, 2, D).transpose(1, 0, 2)
    w3s = W3.reshape(1, 2 * D, D)

    h1 = _c1(x_p, W1, deg)
    agg1 = _agg(h1, srcc, dst2d)
    h2 = _c2(agg1, agg1, w2s, b1.reshape(2, D), deg)
    agg2 = _agg(h2, srcc, dst2d)
    h3 = _c3(agg2, agg2, w3s, b2.reshape(2, D), deg)
    agg3 = _agg(h3, srcc, dst2d)
    out3, out3w = _c4(agg3, deg, b3.reshape(1, D), Wp.reshape(1, D))
    hs, hd = _label_kernel(out3w, out3, e0_2d, e1_2d)
    pred3d = _c5(hs, hd, bp.reshape(1, 1))
    return pred3d.reshape(L_PAD)[:n_l]
